# baseline (speedup 1.0000x reference)
import functools

import jax
import jax.numpy as jnp
from jax import lax
from jax.experimental import pallas as pl
from jax.experimental.pallas import tpu as pltpu
from jax.experimental.pallas import tpu_sc as plsc

B = 16
S = 512
D = 768
L = 16
NV = D // L
EPS = 1e-12

_info = plsc.get_sparse_core_info()
NC = _info.num_cores
NS = _info.num_subcores
NW = NC * NS

S_PER_W = S // NW
TOK_PER_W = B * S_PER_W
CH = 32
NCH = TOK_PER_W // CH
NBUF = 4
SL_PER_CH = CH // B


_GATHER_DNUMS = lax.GatherDimensionNumbers(
    offset_dims=(), collapsed_slice_dims=(0,), start_index_map=(0,))


def _lane_shuffle(v, perm):
    return lax.gather(v, perm.reshape(L, 1), _GATHER_DNUMS, slice_sizes=(1,),
                      mode=lax.GatherScatterMode.PROMISE_IN_BOUNDS)


def _allreduce_sum(v):
    for k in (8, 4, 2, 1):
        perm = lax.iota(jnp.int32, L) ^ k
        v = v + _lane_shuffle(v, perm)
    return v


def _rsqrt_vec(x):
    i = plsc.bitcast(x, jnp.int32)
    i = jnp.int32(0x5F3759DF) - lax.shift_right_logical(i, 1)
    y = plsc.bitcast(i, jnp.float32)
    for _ in range(3):
        y = y * (jnp.float32(1.5) - jnp.float32(0.5) * x * y * y)
    return y


@functools.partial(
    pl.kernel,
    out_type=jax.ShapeDtypeStruct((B * S, D), jnp.float32),
    mesh=plsc.VectorSubcoreMesh(core_axis_name="c", subcore_axis_name="s"),
    compiler_params=pltpu.CompilerParams(needs_layout_passes=False),
    scratch_types=(
        [
            pltpu.VMEM((NCH, CH), jnp.int32),
            pltpu.VMEM((NCH, CH), jnp.int32),
            pltpu.VMEM((S_PER_W, D), jnp.float32),
            pltpu.VMEM((D,), jnp.float32),
            pltpu.VMEM((D,), jnp.float32),
        ]
        + [pltpu.VMEM((CH, D), jnp.float32) for _ in range(NBUF)]
        + [pltpu.SemaphoreType.DMA for _ in range(2 * NBUF)]
    ),
)
def _embed_ln(ids_hbm, tok_hbm, pos_hbm, gam_hbm, bet_hbm, out_hbm,
              idx_v, dst_v, pos_v, g_v, b_v, *rest):
    bufs = list(rest[:NBUF])
    gsem = list(rest[NBUF:2 * NBUF])
    ssem = list(rest[2 * NBUF:])

    w = lax.axis_index("s") * NC + lax.axis_index("c")
    base_s = w * S_PER_W

    pltpu.sync_copy(ids_hbm.at[w], idx_v)
    pltpu.sync_copy(pos_hbm.at[pl.ds(base_s, S_PER_W)], pos_v)
    pltpu.sync_copy(gam_hbm, g_v)
    pltpu.sync_copy(bet_hbm, b_v)

    lane = lax.iota(jnp.int32, L)
    for c in range(NCH):
        for g in range(SL_PER_CH):
            s_abs = base_s + c * SL_PER_CH + g
            dst_v[c, pl.ds(g * L, L)] = lane * S + s_abs

    def start_gather(c):
        return pltpu.async_copy(
            tok_hbm.at[idx_v.at[c]], bufs[c % NBUF], gsem[c % NBUF])

    def start_scatter(c):
        return pltpu.async_copy(
            bufs[c % NBUF], out_hbm.at[dst_v.at[c]], ssem[c % NBUF])

    def compute_chunk(c):
        buf = bufs[c % NBUF]

        def row_body(r, _):
            p = c * SL_PER_CH + r // B
            acc_s = jnp.zeros((L,), jnp.float32)
            acc_q = jnp.zeros((L,), jnp.float32)
            for j in range(NV):
                x = buf[r, pl.ds(j * L, L)] + pos_v[p, pl.ds(j * L, L)]
                buf[r, pl.ds(j * L, L)] = x
                acc_s = acc_s + x
                acc_q = acc_q + x * x
            mean_v = _allreduce_sum(acc_s) * jnp.float32(1.0 / D)
            msq_v = _allreduce_sum(acc_q) * jnp.float32(1.0 / D)
            var_v = jnp.maximum(msq_v - mean_v * mean_v, jnp.float32(0.0))
            rstd = _rsqrt_vec(var_v + jnp.float32(EPS))
            for j in range(NV):
                x = buf[r, pl.ds(j * L, L)]
                y = (x - mean_v) * rstd
                buf[r, pl.ds(j * L, L)] = (
                    y * g_v[pl.ds(j * L, L)] + b_v[pl.ds(j * L, L)])
            return 0

        lax.fori_loop(0, CH, row_body, 0)

    ghandles = {}
    shandles = {}
    for c in range(min(NBUF - 1, NCH)):
        ghandles[c] = start_gather(c)
    for c in range(NCH):
        ghandles[c].wait()
        compute_chunk(c)
        shandles[c] = start_scatter(c)
        n = c + NBUF - 1
        if n < NCH:
            if n - NBUF >= 0:
                shandles[n - NBUF].wait()
            ghandles[n] = start_gather(n)
    for c in range(max(0, NCH - NBUF), NCH):
        shandles[c].wait()


def kernel(input_ids, token_table, pos_table, ln_gamma, ln_beta):
    ids_g = jnp.transpose(input_ids).reshape(NW, NCH, CH)
    out = _embed_ln(ids_g, token_table, pos_table, ln_gamma, ln_beta)
    return out.reshape(B, S, D)


# speedup vs baseline: 2.8332x; 2.8332x over previous
import functools

import jax
import jax.numpy as jnp
from jax import lax
from jax.experimental import pallas as pl
from jax.experimental.pallas import tpu as pltpu
from jax.experimental.pallas import tpu_sc as plsc

B = 16
S = 512
D = 768
BS = B * S
EPS = 1e-12

_info = plsc.get_sparse_core_info()
NC = _info.num_cores
NS = _info.num_subcores
NW = NC * NS

TOK_PER_W = BS // NW
CH = 64
NCH = TOK_PER_W // CH
NBUF = 2

TBLK = 512


@functools.partial(
    pl.kernel,
    out_type=jax.ShapeDtypeStruct((BS, D), jnp.float32),
    mesh=plsc.VectorSubcoreMesh(core_axis_name="c", subcore_axis_name="s"),
    compiler_params=pltpu.CompilerParams(needs_layout_passes=False),
    scratch_types=(
        [pltpu.VMEM((NCH, CH), jnp.int32)]
        + [pltpu.VMEM((CH, D), jnp.float32) for _ in range(NBUF)]
        + [pltpu.SemaphoreType.DMA for _ in range(2 * NBUF)]
    ),
)
def _gather_sc(ids_hbm, tok_hbm, out_hbm, idx_v, *rest):
    bufs = list(rest[:NBUF])
    gsem = list(rest[NBUF:2 * NBUF])
    ssem = list(rest[2 * NBUF:])

    w = lax.axis_index("s") * NC + lax.axis_index("c")
    base = w * TOK_PER_W

    pltpu.sync_copy(ids_hbm.at[w], idx_v)

    def start_gather(c):
        return pltpu.async_copy(
            tok_hbm.at[idx_v.at[c]], bufs[c % NBUF], gsem[c % NBUF])

    def start_out(c):
        return pltpu.async_copy(
            bufs[c % NBUF], out_hbm.at[pl.ds(base + c * CH, CH)],
            ssem[c % NBUF])

    ghandles = {}
    shandles = {}
    for c in range(min(NBUF, NCH)):
        ghandles[c] = start_gather(c)
    for c in range(NCH):
        ghandles[c].wait()
        shandles[c] = start_out(c)
        n = c + NBUF
        if n < NCH:
            shandles[n - NBUF].wait()
            ghandles[n] = start_gather(n)
    for c in range(max(0, NCH - NBUF), NCH):
        shandles[c].wait()


def _ln_body(x_ref, pos_ref, g_ref, b_ref, o_ref):
    e = x_ref[...] + pos_ref[...]
    mean = jnp.mean(e, axis=1, keepdims=True)
    var = jnp.mean(jnp.square(e - mean), axis=1, keepdims=True)
    y = (e - mean) * lax.rsqrt(var + EPS)
    o_ref[...] = y * g_ref[...] + b_ref[...]


_ln_tc = pl.pallas_call(
    _ln_body,
    grid=(BS // TBLK,),
    in_specs=[
        pl.BlockSpec((TBLK, D), lambda i: (i, 0)),
        pl.BlockSpec((S, D), lambda i: (0, 0)),
        pl.BlockSpec((1, D), lambda i: (0, 0)),
        pl.BlockSpec((1, D), lambda i: (0, 0)),
    ],
    out_specs=pl.BlockSpec((TBLK, D), lambda i: (i, 0)),
    out_shape=jax.ShapeDtypeStruct((BS, D), jnp.float32),
)


def kernel(input_ids, token_table, pos_table, ln_gamma, ln_beta):
    ids_g = input_ids.reshape(NW, NCH, CH)
    emb = _gather_sc(ids_g, token_table)
    out = _ln_tc(emb, pos_table, ln_gamma.reshape(1, D), ln_beta.reshape(1, D))
    return out.reshape(B, S, D)
